# baseline (speedup 1.0000x reference)
import functools

import jax
import jax.numpy as jnp
from jax import lax
from jax.experimental import pallas as pl
from jax.experimental.pallas import tpu as pltpu
from jax.experimental.pallas import tpu_sc as plsc

NC = 2
NS = 16
NW = NC * NS
CH = 128
K = 4


@functools.partial(jax.jit, static_argnums=(2, 3))
def _gather_call(idx2d, table, n_rows, dim):
    n_chunks = n_rows // CH
    cw = n_chunks // NW
    nb = cw // K
    mesh = plsc.VectorSubcoreMesh(core_axis_name="c", subcore_axis_name="s")

    @functools.partial(
        pl.kernel,
        mesh=mesh,
        out_type=jax.ShapeDtypeStruct((n_rows, dim), jnp.float32),
        scratch_types=[
            pltpu.VMEM((cw, CH), jnp.int32),
            pltpu.VMEM((K, CH, dim), jnp.float32),
            pltpu.SemaphoreType.DMA,
            pltpu.SemaphoreType.DMA,
        ],
        compiler_params=pltpu.CompilerParams(use_tc_tiling_on_sc=False),
    )
    def body(idx_hbm, table_hbm, out_hbm, idx_v, buf, gsem, osem):
        wid = lax.axis_index("s") * NC + lax.axis_index("c")
        c0 = wid * cw
        pltpu.sync_copy(idx_hbm.at[pl.ds(c0, cw)], idx_v)

        def batch(g, _):
            gathers = []
            for b in range(K):
                c = g * K + b
                gathers.append(
                    pltpu.async_copy(
                        table_hbm.at[idx_v.at[c]], buf.at[b], gsem
                    )
                )
            for h in gathers:
                h.wait()
            writes = []
            for b in range(K):
                r = (c0 + g * K + b) * CH
                writes.append(
                    pltpu.async_copy(
                        buf.at[b], out_hbm.at[pl.ds(r, CH)], osem
                    )
                )
            for h in writes:
                h.wait()
            return _

        lax.fori_loop(0, nb, batch, None)

    return body(idx2d, table)


def kernel(x, table):
    bsz, seq = x.shape
    dim = table.shape[1]
    n_rows = bsz * seq
    idx2d = x.astype(jnp.int32).reshape(n_rows // CH, CH)
    out = _gather_call(idx2d, table, n_rows, dim)
    return out.reshape(bsz, seq, dim)


# speedup vs baseline: 1.0103x; 1.0103x over previous
import functools

import jax
import jax.numpy as jnp
from jax import lax
from jax.experimental import pallas as pl
from jax.experimental.pallas import tpu as pltpu
from jax.experimental.pallas import tpu_sc as plsc

NC = 2
NS = 16
NW = NC * NS
CH = 128
D = 8


@functools.partial(jax.jit, static_argnums=(2, 3))
def _gather_call(idx2d, table, n_rows, dim):
    n_chunks = n_rows // CH
    cw = n_chunks // NW
    mesh = plsc.VectorSubcoreMesh(core_axis_name="c", subcore_axis_name="s")

    @functools.partial(
        pl.kernel,
        mesh=mesh,
        out_type=jax.ShapeDtypeStruct((n_rows, dim), jnp.float32),
        scratch_types=[
            pltpu.VMEM((cw, CH), jnp.int32),
            pltpu.VMEM((D, CH, dim), jnp.float32),
            [pltpu.SemaphoreType.DMA] * D,
            [pltpu.SemaphoreType.DMA] * D,
        ],
        compiler_params=pltpu.CompilerParams(use_tc_tiling_on_sc=False),
    )
    def body(idx_hbm, table_hbm, out_hbm, idx_v, buf, gsems, osems):
        wid = lax.axis_index("s") * NC + lax.axis_index("c")
        c0 = wid * cw
        pltpu.sync_copy(idx_hbm.at[pl.ds(c0, cw)], idx_v)

        def issue_gather(c, s):
            pltpu.async_copy(table_hbm.at[idx_v.at[c]], buf.at[s], gsems[s])

        def drain_gather_issue_write(c, s):
            pltpu.make_async_copy(
                table_hbm.at[pl.ds(0, CH)], buf.at[s], gsems[s]
            ).wait()
            r = (c0 + c) * CH
            pltpu.async_copy(buf.at[s], out_hbm.at[pl.ds(r, CH)], osems[s])

        def wait_write(s):
            pltpu.make_async_copy(
                buf.at[s], out_hbm.at[pl.ds(0, CH)], osems[s]
            ).wait()

        def ring_pass(i, _):
            for b in range(D):
                c = i * D + b

                @pl.when(i > 0)
                def _():
                    wait_write(b)

                issue_gather(c, b)
                if b == 0:
                    @pl.when(i > 0)
                    def _():
                        drain_gather_issue_write(c - 1, D - 1)
                else:
                    drain_gather_issue_write(c - 1, b - 1)
            return _

        lax.fori_loop(0, cw // D, ring_pass, None)
        drain_gather_issue_write(cw - 1, D - 1)
        for s in range(D):
            wait_write(s)

    return body(idx2d, table)


def kernel(x, table):
    bsz, seq = x.shape
    dim = table.shape[1]
    n_rows = bsz * seq
    idx2d = x.astype(jnp.int32).reshape(n_rows // CH, CH)
    out = _gather_call(idx2d, table, n_rows, dim)
    return out.reshape(bsz, seq, dim)


# speedup vs baseline: 1.0112x; 1.0008x over previous
import functools

import jax
import jax.numpy as jnp
from jax import lax
from jax.experimental import pallas as pl
from jax.experimental.pallas import tpu as pltpu
from jax.experimental.pallas import tpu_sc as plsc

NC = 2
NS = 16
NW = NC * NS
CH = 128
D = 8


@functools.partial(jax.jit, static_argnums=(2, 3))
def _gather_call(idx2d, table, n_rows, dim):
    n_chunks = n_rows // CH
    cw = n_chunks // NW
    mesh = plsc.VectorSubcoreMesh(core_axis_name="c", subcore_axis_name="s")

    @functools.partial(
        pl.kernel,
        mesh=mesh,
        out_type=jax.ShapeDtypeStruct((n_rows, dim), jnp.float32),
        scratch_types=[
            pltpu.VMEM((cw, CH), jnp.int32),
            pltpu.VMEM((D, CH, dim), jnp.float32),
            [pltpu.SemaphoreType.DMA] * D,
            [pltpu.SemaphoreType.DMA] * D,
        ],
        compiler_params=pltpu.CompilerParams(use_tc_tiling_on_sc=False),
    )
    def body(idx_hbm, table_hbm, out_hbm, idx_v, buf, gsems, osems):
        wid = lax.axis_index("s") * NC + lax.axis_index("c")
        c0 = wid * cw
        pltpu.sync_copy(idx_hbm.at[pl.ds(c0, cw)], idx_v)

        def issue_gather(c, s):
            pltpu.async_copy(table_hbm.at[idx_v.at[c]], buf.at[s], gsems[s])

        def drain_gather_issue_write(c, s):
            pltpu.make_async_copy(
                table_hbm.at[idx_v.at[c]], buf.at[s], gsems[s]
            ).wait()
            r = (c0 + c) * CH
            pltpu.async_copy(buf.at[s], out_hbm.at[pl.ds(r, CH)], osems[s])

        def wait_write(s):
            pltpu.make_async_copy(
                buf.at[s], out_hbm.at[pl.ds(0, CH)], osems[s]
            ).wait()

        def ring_pass(i, _):
            for b in range(D):
                c = i * D + b

                @pl.when(i > 0)
                def _():
                    wait_write(b)

                issue_gather(c, b)
                if b == 0:
                    @pl.when(i > 0)
                    def _():
                        drain_gather_issue_write(c - 1, D - 1)
                else:
                    drain_gather_issue_write(c - 1, b - 1)
            return _

        lax.fori_loop(0, cw // D, ring_pass, None)
        drain_gather_issue_write(cw - 1, D - 1)
        for s in range(D):
            wait_write(s)

    return body(idx2d, table)


def kernel(x, table):
    bsz, seq = x.shape
    dim = table.shape[1]
    n_rows = bsz * seq
    idx2d = x.astype(jnp.int32).reshape(n_rows // CH, CH)
    out = _gather_call(idx2d, table, n_rows, dim)
    return out.reshape(bsz, seq, dim)
